# Initial kernel scaffold; baseline (speedup 1.0000x reference)
import jax
import jax.numpy as jnp
from jax.experimental import pallas as pl


def kernel(x_job, x_station, x_machine, x_robot, ei_can_load, ei_loaded, ei_will_execute, ei_execute, ei_hold, W, att_src, att_dst, bias, ln_gamma, ln_beta):
    raise NotImplementedError("write your pallas kernel here")


# speedup vs baseline: 24.9620x; 24.9620x over previous
import dataclasses
import functools

import jax
import jax.numpy as jnp
from jax import lax
from jax.experimental import pallas as pl
from jax.experimental.pallas import tpu as pltpu
from jax.experimental.pallas import tpu_sc as plsc

N_JOB = 50000
N_SRC = 10000
E = 120000
D = 128
H = 4
C = 32
NREL = 5

NC = 2
NS = 16
TILES = NC * NS

E_PAD = 122880
SLAB = E_PAD // TILES
CH = 640

DEN_R = 250240
DEN_PR = 51200
DEN_PSLAB = DEN_PR // NS
SLAB2 = E_PAD // NS
ACC_ROWS = 50048
ACC_SLAB = ACC_ROWS // NS


def _mesh():
    return plsc.VectorSubcoreMesh(core_axis_name="c", subcore_axis_name="s")


def _sc_params():
    cp = pltpu.CompilerParams()
    if "needs_layout_passes" in pltpu.CompilerParams.__dataclass_fields__:
        cp = dataclasses.replace(cp, needs_layout_passes=False)
    if "use_tc_tiling_on_sc" in pltpu.CompilerParams.__dataclass_fields__:
        cp = dataclasses.replace(cp, use_tc_tiling_on_sc=False)
    return cp


def _tc_project(xs, W, As):
    BLK = 2000

    def body(x_ref, w_ref, a_ref, hs_ref, as_ref):
        h = jnp.dot(x_ref[0], w_ref[0], preferred_element_type=jnp.float32)
        hs_ref[0] = h
        as_ref[0] = jnp.dot(h, a_ref[0], preferred_element_type=jnp.float32)

    return pl.pallas_call(
        body,
        grid=(NREL, N_SRC // BLK),
        in_specs=[
            pl.BlockSpec((1, BLK, D), lambda r, i: (r, i, 0)),
            pl.BlockSpec((1, D, D), lambda r, i: (r, 0, 0)),
            pl.BlockSpec((1, D, H), lambda r, i: (r, 0, 0)),
        ],
        out_specs=[
            pl.BlockSpec((1, BLK, D), lambda r, i: (r, i, 0)),
            pl.BlockSpec((1, BLK, H), lambda r, i: (r, i, 0)),
        ],
        out_shape=[
            jax.ShapeDtypeStruct((NREL, N_SRC, D), jnp.float32),
            jax.ShapeDtypeStruct((NREL, N_SRC, H), jnp.float32),
        ],
    )(xs, W, As)


def _tc_dst_alpha(x_job, W, Ad):
    BLK = 2000

    def body(x_ref, w_ref, ad_ref, out_ref):
        cols = [
            jnp.dot(w_ref[r], ad_ref[r], preferred_element_type=jnp.float32)
            for r in range(NREL)
        ]
        wd = jnp.concatenate(cols, axis=1)
        out_ref[...] = jnp.dot(x_ref[...], wd, preferred_element_type=jnp.float32)

    return pl.pallas_call(
        body,
        grid=(N_JOB // BLK,),
        in_specs=[
            pl.BlockSpec((BLK, D), lambda i: (i, 0)),
            pl.BlockSpec((NREL, D, D), lambda i: (0, 0, 0)),
            pl.BlockSpec((NREL, D, 16), lambda i: (0, 0, 0)),
        ],
        out_specs=pl.BlockSpec((BLK, 16 * NREL), lambda i: (i, 0)),
        out_shape=jax.ShapeDtypeStruct((N_JOB, 16 * NREL), jnp.float32),
    )(x_job, W, Ad)


def _sc_edge_ex_den(src_flat, dst_flat, asv, adv_flat, zden):
    @functools.partial(
        pl.kernel,
        out_type=[
            jax.ShapeDtypeStruct((NREL * E_PAD, H), jnp.float32),
            jax.ShapeDtypeStruct((NREL * DEN_PR, 8), jnp.float32),
        ],
        mesh=_mesh(),
        compiler_params=_sc_params(),
        scratch_types=[
            pltpu.VMEM((N_SRC, H), jnp.float32),
            pltpu.VMEM((CH,), jnp.int32),
            pltpu.VMEM((CH,), jnp.int32),
            pltpu.VMEM((CH,), jnp.int32),
            pltpu.VMEM((CH, 16), jnp.float32),
            pltpu.VMEM((CH, H), jnp.float32),
            pltpu.VMEM((CH, 8), jnp.float32),
            pltpu.VMEM_SHARED((DEN_PR, 8), jnp.float32),
        ],
    )
    def k(src_hbm, dst_hbm, as_hbm, ad_hbm, zden_hbm, ex_hbm, den_hbm,
          as_tab, srcv, dstv, adix, adrows, exb, exb8, den_sp):
        core = lax.axis_index("c")
        sub = lax.axis_index("s")
        iota16 = lax.iota(jnp.int32, 16)

        rlo = core * 2
        rhi = 2 + core * 3

        pltpu.sync_copy(zden_hbm.at[pl.ds(0, CH)], exb8)

        @pl.loop(rlo, rhi)
        def _(r):
            pltpu.sync_copy(as_hbm.at[r], as_tab)
            pltpu.sync_copy(zden_hbm, den_sp.at[pl.ds(sub * DEN_PSLAB, DEN_PSLAB)])
            plsc.subcore_barrier()

            @pl.loop(0, SLAB2 // CH)
            def _(chix):
                base = r * E_PAD + sub * SLAB2 + chix * CH
                pltpu.sync_copy(src_hbm.at[pl.ds(base, CH)], srcv)
                pltpu.sync_copy(dst_hbm.at[pl.ds(base, CH)], dstv)

                @pl.loop(0, CH, step=16)
                def _(i):
                    adix[pl.ds(i, 16)] = dstv[pl.ds(i, 16)] * 5 + r

                pltpu.sync_copy(ad_hbm.at[adix], adrows)

                epos0 = sub * SLAB2 + chix * CH

                @pl.loop(0, CH, step=16)
                def _(i):
                    s16 = srcv[pl.ds(i, 16)]
                    pos = iota16 + (epos0 + i)
                    mask = pos < E
                    off = iota16 + i
                    for h in range(H):
                        hvec = jnp.full((16,), h, jnp.int32)
                        a_s = plsc.load_gather(as_tab, [s16, hvec])
                        a_d = plsc.load_gather(adrows, [off, hvec])
                        s = a_s + a_d
                        s = jnp.maximum(s, 0.2 * s)
                        ex = jnp.where(mask, jnp.exp(s), 0.0)
                        plsc.store_scatter(exb, [off, hvec], ex)
                        plsc.store_scatter(exb8, [off, hvec], ex)

                pltpu.sync_copy(exb, ex_hbm.at[pl.ds(base, CH)])
                pltpu.sync_copy(exb8, den_sp.at[dstv], add=True)

            plsc.subcore_barrier()
            pltpu.sync_copy(
                den_sp.at[pl.ds(sub * DEN_PSLAB, DEN_PSLAB)],
                den_hbm.at[pl.ds(r * DEN_PR + sub * DEN_PSLAB, DEN_PSLAB)],
            )
            plsc.subcore_barrier()

    return k(src_flat, dst_flat, asv, adv_flat, zden)


def _tc_invden(den2):
    BLK = 400

    def body(d_ref, out_ref):
        inv = 1.0 / (d_ref[...] + 1e-16)
        out_ref[...] = jnp.concatenate(
            [inv, jnp.zeros((BLK, 8), jnp.float32)], axis=-1)

    def in_map(r, i):
        return (r * (DEN_PR // BLK) + i, 0)

    def out_map(r, i):
        return (r * (N_JOB // BLK) + i, 0)

    return pl.pallas_call(
        body,
        grid=(NREL, N_JOB // BLK),
        in_specs=[pl.BlockSpec((BLK, 8), in_map)],
        out_specs=pl.BlockSpec((BLK, 16), out_map),
        out_shape=jax.ShapeDtypeStruct((DEN_R, 16), jnp.float32),
    )(den2)


def _sc_coef(dst_flat, ex, invden):
    @functools.partial(
        pl.kernel,
        out_type=jax.ShapeDtypeStruct((NREL * H * E_PAD,), jnp.float32),
        mesh=_mesh(),
        compiler_params=_sc_params(),
        scratch_types=[
            pltpu.VMEM((CH,), jnp.int32),
            pltpu.VMEM((CH,), jnp.int32),
            pltpu.VMEM((CH, H), jnp.float32),
            pltpu.VMEM((CH, 16), jnp.float32),
            pltpu.VMEM((H, CH), jnp.float32),
        ],
    )
    def k(dst_hbm, ex_hbm, inv_hbm, coef_hbm, dstv, dnix, exrows, invrows, cbuf):
        core = lax.axis_index("c")
        sub = lax.axis_index("s")
        w = core * NS + sub
        iota16 = lax.iota(jnp.int32, 16)

        @pl.loop(0, NREL)
        def _(r):
            @pl.loop(0, SLAB // CH)
            def _(chix):
                base = r * E_PAD + w * SLAB + chix * CH
                pltpu.sync_copy(dst_hbm.at[pl.ds(base, CH)], dstv)
                pltpu.sync_copy(ex_hbm.at[pl.ds(base, CH)], exrows)

                @pl.loop(0, CH, step=16)
                def _(i):
                    dnix[pl.ds(i, 16)] = dstv[pl.ds(i, 16)] + r * N_JOB

                pltpu.sync_copy(inv_hbm.at[dnix], invrows)

                @pl.loop(0, CH, step=16)
                def _(i):
                    off = iota16 + i
                    for h in range(H):
                        hvec = jnp.full((16,), h, jnp.int32)
                        exv = plsc.load_gather(exrows, [off, hvec])
                        inv = plsc.load_gather(invrows, [off, hvec])
                        cbuf[h, pl.ds(i, 16)] = exv * inv

                for h in range(H):
                    pltpu.sync_copy(
                        cbuf.at[h],
                        coef_hbm.at[pl.ds(
                            (r * H + h) * E_PAD + w * SLAB + chix * CH, CH)],
                    )

    return k(dst_flat, ex, invden)


def _sc_messages(src_flat, dst_flat, coef, hs_flat, zacc):
    CH2 = C // 2

    @functools.partial(
        pl.kernel,
        out_type=jax.ShapeDtypeStruct((2 * H * ACC_ROWS, CH2), jnp.float32),
        mesh=_mesh(),
        compiler_params=_sc_params(),
        scratch_types=[
            pltpu.VMEM((CH,), jnp.int32),
            pltpu.VMEM((CH,), jnp.int32),
            pltpu.VMEM((CH,), jnp.int32),
            pltpu.VMEM((CH,), jnp.float32),
            pltpu.VMEM((CH, CH2), jnp.float32),
            pltpu.VMEM_SHARED((ACC_ROWS, CH2), jnp.float32),
        ],
    )
    def k(src_hbm, dst_hbm, coef_hbm, hs_hbm, zacc_hbm, num_hbm,
          srcv, dstv, hsix, coefv, hrows, acc):
        core = lax.axis_index("c")
        sub = lax.axis_index("s")
        w = core * NS + sub

        @pl.loop(0, 4)
        def _(pi):
            h = core * 2 + (pi >> 1)
            half = pi & 1
            p = core * 4 + pi
            pltpu.sync_copy(zacc_hbm, acc.at[pl.ds(sub * ACC_SLAB, ACC_SLAB)])
            plsc.subcore_barrier()

            @pl.loop(0, NREL)
            def _(r):
                @pl.loop(0, SLAB2 // CH)
                def _(chix):
                    base = r * E_PAD + sub * SLAB2 + chix * CH
                    pltpu.sync_copy(src_hbm.at[pl.ds(base, CH)], srcv)
                    pltpu.sync_copy(dst_hbm.at[pl.ds(base, CH)], dstv)
                    cbase = (r * H + h) * E_PAD + sub * SLAB2 + chix * CH
                    pltpu.sync_copy(coef_hbm.at[pl.ds(cbase, CH)], coefv)

                    hs0 = r * N_SRC * 2 * H + h * 2 + half

                    @pl.loop(0, CH, step=16)
                    def _(i):
                        hsix[pl.ds(i, 16)] = srcv[pl.ds(i, 16)] * (2 * H) + hs0

                    pltpu.sync_copy(hs_hbm.at[hsix], hrows)

                    @pl.loop(0, CH, step=16)
                    def _(i):
                        c16 = coefv[pl.ds(i, 16)]
                        for j in range(16):
                            cv = c16[j]
                            hrows[i + j, :] = hrows[i + j, :] * cv

                    pltpu.sync_copy(hrows, acc.at[dstv], add=True)

            plsc.subcore_barrier()
            pltpu.sync_copy(
                acc.at[pl.ds(sub * ACC_SLAB, ACC_SLAB)],
                num_hbm.at[pl.ds(p * ACC_ROWS + sub * ACC_SLAB, ACC_SLAB)],
            )
            plsc.subcore_barrier()

    return k(src_flat, dst_flat, coef, hs_flat, zacc)


def _tc_epilogue(parts, x_job, bias, ln_gamma, ln_beta):
    BLK = 1000

    W16 = C // 2

    def body(*refs):
        nrefs = refs[:8]
        xr, br, gr, btr, outr = refs[8:]
        bsum = jnp.sum(br[...], axis=0, keepdims=True)
        phs = []
        for q, nr in enumerate(nrefs):
            ph = (nr[...] + xr[:, q * W16:(q + 1) * W16]
                  + bsum[:, q * W16:(q + 1) * W16])
            phs.append(jnp.maximum(ph, 0.0))
        s1 = phs[0].sum(-1, keepdims=True)
        for p in phs[1:]:
            s1 = s1 + p.sum(-1, keepdims=True)
        mu = s1 * (1.0 / D)
        s2 = ((phs[0] - mu) ** 2).sum(-1, keepdims=True)
        for p in phs[1:]:
            s2 = s2 + ((p - mu) ** 2).sum(-1, keepdims=True)
        rstd = lax.rsqrt(s2 * (1.0 / D) + 1e-5)
        for q, p in enumerate(phs):
            outr[:, q * W16:(q + 1) * W16] = (
                (p - mu) * rstd * gr[:, q * W16:(q + 1) * W16]
                + btr[:, q * W16:(q + 1) * W16]
            )

    return pl.pallas_call(
        body,
        grid=(N_JOB // BLK,),
        in_specs=[pl.BlockSpec((BLK, W16), lambda i: (i, 0))] * 8 + [
            pl.BlockSpec((BLK, D), lambda i: (i, 0)),
            pl.BlockSpec((NREL, D), lambda i: (0, 0)),
            pl.BlockSpec((1, D), lambda i: (0, 0)),
            pl.BlockSpec((1, D), lambda i: (0, 0)),
        ],
        out_specs=pl.BlockSpec((BLK, D), lambda i: (i, 0)),
        out_shape=jax.ShapeDtypeStruct((N_JOB, D), jnp.float32),
    )(*parts, x_job, bias, ln_gamma.reshape(1, D), ln_beta.reshape(1, D))


def kernel(x_job, x_station, x_machine, x_robot, ei_can_load, ei_loaded,
           ei_will_execute, ei_execute, ei_hold, W, att_src, att_dst, bias,
           ln_gamma, ln_beta):
    eis = [ei_can_load, ei_loaded, ei_will_execute, ei_execute, ei_hold]
    xs = jnp.stack([x_station, x_station, x_machine, x_machine, x_robot])

    srcs = [jnp.pad(ei[0].astype(jnp.int32), (0, E_PAD - E)) for ei in eis]
    dsts = [jnp.pad(ei[1].astype(jnp.int32), (0, E_PAD - E)) for ei in eis]
    src_flat = jnp.concatenate(srcs)
    dst_flat = jnp.concatenate(dsts)

    eye = jnp.eye(H, dtype=jnp.float32)
    As = (att_src[:, :, :, None] * eye[:, None, :]).reshape(NREL, D, H)
    Ad4 = (att_dst[:, :, :, None] * eye[:, None, :]).reshape(NREL, D, H)
    Ad = jnp.concatenate([Ad4, jnp.zeros((NREL, D, 12), jnp.float32)], axis=-1)

    hs, asv = _tc_project(xs, W, As)
    adv = _tc_dst_alpha(x_job, W, Ad)

    zden = jnp.zeros((DEN_PSLAB, 8), jnp.float32)
    ex, den2 = _sc_edge_ex_den(
        src_flat, dst_flat, asv, adv.reshape(NREL * N_JOB, 16), zden)
    invden = _tc_invden(den2)
    coef = _sc_coef(dst_flat, ex, invden)

    zacc = jnp.zeros((ACC_SLAB, C // 2), jnp.float32)
    num = _sc_messages(
        src_flat, dst_flat, coef, hs.reshape(NREL * N_SRC * 2 * H, C // 2),
        zacc)

    parts = [lax.slice(num, (p * ACC_ROWS, 0), (p * ACC_ROWS + N_JOB, C // 2))
             for p in range(2 * H)]
    return _tc_epilogue(parts, x_job, bias, ln_gamma, ln_beta)


# speedup vs baseline: 26.2644x; 1.0522x over previous
import dataclasses
import functools

import jax
import jax.numpy as jnp
from jax import lax
from jax.experimental import pallas as pl
from jax.experimental.pallas import tpu as pltpu
from jax.experimental.pallas import tpu_sc as plsc

N_JOB = 50000
N_SRC = 10000
E = 120000
D = 128
H = 4
C = 32
NREL = 5

NC = 2
NS = 16
TILES = NC * NS

E_PAD = 122880
SLAB = E_PAD // TILES
CH = 640
CH5 = 1280

DEN_R = 250240
DEN_PR = 51200
DEN_PSLAB = DEN_PR // NS
SLAB2 = E_PAD // NS
ACC_ROWS = 50048
ACC_SLAB = ACC_ROWS // NS


def _mesh():
    return plsc.VectorSubcoreMesh(core_axis_name="c", subcore_axis_name="s")


def _sc_params():
    cp = pltpu.CompilerParams()
    if "needs_layout_passes" in pltpu.CompilerParams.__dataclass_fields__:
        cp = dataclasses.replace(cp, needs_layout_passes=False)
    if "use_tc_tiling_on_sc" in pltpu.CompilerParams.__dataclass_fields__:
        cp = dataclasses.replace(cp, use_tc_tiling_on_sc=False)
    return cp


def _tc_project(xs, W, As):
    BLK = 2000

    def body(x_ref, w_ref, a_ref, hs_ref, as_ref):
        h = jnp.dot(x_ref[0], w_ref[0], preferred_element_type=jnp.float32)
        hs_ref[0] = h
        as_ref[0] = jnp.dot(h, a_ref[0], preferred_element_type=jnp.float32)

    return pl.pallas_call(
        body,
        grid=(NREL, N_SRC // BLK),
        in_specs=[
            pl.BlockSpec((1, BLK, D), lambda r, i: (r, i, 0)),
            pl.BlockSpec((1, D, D), lambda r, i: (r, 0, 0)),
            pl.BlockSpec((1, D, H), lambda r, i: (r, 0, 0)),
        ],
        out_specs=[
            pl.BlockSpec((1, BLK, D), lambda r, i: (r, i, 0)),
            pl.BlockSpec((1, BLK, H), lambda r, i: (r, i, 0)),
        ],
        out_shape=[
            jax.ShapeDtypeStruct((NREL, N_SRC, D), jnp.float32),
            jax.ShapeDtypeStruct((NREL, N_SRC, H), jnp.float32),
        ],
    )(xs, W, As)


def _tc_dst_alpha(x_job, W, Ad):
    BLK = 2000

    def body(x_ref, w_ref, ad_ref, out_ref):
        cols = [
            jnp.dot(w_ref[r], ad_ref[r], preferred_element_type=jnp.float32)
            for r in range(NREL)
        ]
        wd = jnp.concatenate(cols, axis=1)
        out_ref[...] = jnp.dot(x_ref[...], wd, preferred_element_type=jnp.float32)

    return pl.pallas_call(
        body,
        grid=(N_JOB // BLK,),
        in_specs=[
            pl.BlockSpec((BLK, D), lambda i: (i, 0)),
            pl.BlockSpec((NREL, D, D), lambda i: (0, 0, 0)),
            pl.BlockSpec((NREL, D, 16), lambda i: (0, 0, 0)),
        ],
        out_specs=pl.BlockSpec((BLK, 16 * NREL), lambda i: (i, 0)),
        out_shape=jax.ShapeDtypeStruct((N_JOB, 16 * NREL), jnp.float32),
    )(x_job, W, Ad)


def _sc_edge_ex_den(src_flat, dst_flat, asv, adv_flat, zden):
    @functools.partial(
        pl.kernel,
        out_type=[
            jax.ShapeDtypeStruct((NREL * E_PAD, H), jnp.float32),
            jax.ShapeDtypeStruct((NREL * DEN_PR, 8), jnp.float32),
        ],
        mesh=_mesh(),
        compiler_params=_sc_params(),
        scratch_types=[
            pltpu.VMEM((N_SRC, H), jnp.float32),
            pltpu.VMEM((CH,), jnp.int32),
            pltpu.VMEM((CH,), jnp.int32),
            pltpu.VMEM((CH,), jnp.int32),
            pltpu.VMEM((CH, 16), jnp.float32),
            pltpu.VMEM((CH, H), jnp.float32),
            pltpu.VMEM((CH, 8), jnp.float32),
            pltpu.VMEM_SHARED((DEN_PR, 8), jnp.float32),
        ],
    )
    def k(src_hbm, dst_hbm, as_hbm, ad_hbm, zden_hbm, ex_hbm, den_hbm,
          as_tab, srcv, dstv, adix, adrows, exb, exb8, den_sp):
        core = lax.axis_index("c")
        sub = lax.axis_index("s")
        iota16 = lax.iota(jnp.int32, 16)

        rlo = core * 2
        rhi = 2 + core * 3

        pltpu.sync_copy(zden_hbm.at[pl.ds(0, CH)], exb8)

        @pl.loop(rlo, rhi)
        def _(r):
            pltpu.sync_copy(as_hbm.at[r], as_tab)
            pltpu.sync_copy(zden_hbm, den_sp.at[pl.ds(sub * DEN_PSLAB, DEN_PSLAB)])
            plsc.subcore_barrier()

            @pl.loop(0, SLAB2 // CH)
            def _(chix):
                base = r * E_PAD + sub * SLAB2 + chix * CH
                pltpu.sync_copy(src_hbm.at[pl.ds(base, CH)], srcv)
                pltpu.sync_copy(dst_hbm.at[pl.ds(base, CH)], dstv)

                @pl.loop(0, CH, step=16)
                def _(i):
                    adix[pl.ds(i, 16)] = dstv[pl.ds(i, 16)] * 5 + r

                pltpu.sync_copy(ad_hbm.at[adix], adrows)

                epos0 = sub * SLAB2 + chix * CH

                @pl.loop(0, CH, step=16)
                def _(i):
                    s16 = srcv[pl.ds(i, 16)]
                    pos = iota16 + (epos0 + i)
                    mask = pos < E
                    off = iota16 + i
                    for h in range(H):
                        hvec = jnp.full((16,), h, jnp.int32)
                        a_s = plsc.load_gather(as_tab, [s16, hvec])
                        a_d = plsc.load_gather(adrows, [off, hvec])
                        s = a_s + a_d
                        s = jnp.maximum(s, 0.2 * s)
                        ex = jnp.where(mask, jnp.exp(s), 0.0)
                        plsc.store_scatter(exb, [off, hvec], ex)
                        plsc.store_scatter(exb8, [off, hvec], ex)

                pltpu.sync_copy(exb, ex_hbm.at[pl.ds(base, CH)])
                pltpu.sync_copy(exb8, den_sp.at[dstv], add=True)

            plsc.subcore_barrier()
            pltpu.sync_copy(
                den_sp.at[pl.ds(sub * DEN_PSLAB, DEN_PSLAB)],
                den_hbm.at[pl.ds(r * DEN_PR + sub * DEN_PSLAB, DEN_PSLAB)],
            )
            plsc.subcore_barrier()

    return k(src_flat, dst_flat, asv, adv_flat, zden)


def _tc_invden(den2):
    BLK = 400

    def body(d_ref, out_ref):
        inv = 1.0 / (d_ref[...] + 1e-16)
        out_ref[...] = jnp.concatenate(
            [inv, jnp.zeros((BLK, 8), jnp.float32)], axis=-1)

    def in_map(r, i):
        return (r * (DEN_PR // BLK) + i, 0)

    def out_map(r, i):
        return (r * (N_JOB // BLK) + i, 0)

    return pl.pallas_call(
        body,
        grid=(NREL, N_JOB // BLK),
        in_specs=[pl.BlockSpec((BLK, 8), in_map)],
        out_specs=pl.BlockSpec((BLK, 16), out_map),
        out_shape=jax.ShapeDtypeStruct((DEN_R, 16), jnp.float32),
    )(den2)


def _sc_coef(dst_flat, ex, invden):
    @functools.partial(
        pl.kernel,
        out_type=jax.ShapeDtypeStruct((NREL * H * E_PAD,), jnp.float32),
        mesh=_mesh(),
        compiler_params=_sc_params(),
        scratch_types=[
            pltpu.VMEM((CH,), jnp.int32),
            pltpu.VMEM((CH,), jnp.int32),
            pltpu.VMEM((CH, H), jnp.float32),
            pltpu.VMEM((CH, 16), jnp.float32),
            pltpu.VMEM((H, CH), jnp.float32),
        ],
    )
    def k(dst_hbm, ex_hbm, inv_hbm, coef_hbm, dstv, dnix, exrows, invrows, cbuf):
        core = lax.axis_index("c")
        sub = lax.axis_index("s")
        w = core * NS + sub
        iota16 = lax.iota(jnp.int32, 16)

        @pl.loop(0, NREL)
        def _(r):
            @pl.loop(0, SLAB // CH)
            def _(chix):
                base = r * E_PAD + w * SLAB + chix * CH
                pltpu.sync_copy(dst_hbm.at[pl.ds(base, CH)], dstv)
                pltpu.sync_copy(ex_hbm.at[pl.ds(base, CH)], exrows)

                @pl.loop(0, CH, step=16)
                def _(i):
                    dnix[pl.ds(i, 16)] = dstv[pl.ds(i, 16)] + r * N_JOB

                pltpu.sync_copy(inv_hbm.at[dnix], invrows)

                @pl.loop(0, CH, step=16)
                def _(i):
                    off = iota16 + i
                    for h in range(H):
                        hvec = jnp.full((16,), h, jnp.int32)
                        exv = plsc.load_gather(exrows, [off, hvec])
                        inv = plsc.load_gather(invrows, [off, hvec])
                        cbuf[h, pl.ds(i, 16)] = exv * inv

                for h in range(H):
                    pltpu.sync_copy(
                        cbuf.at[h],
                        coef_hbm.at[pl.ds(
                            (r * H + h) * E_PAD + w * SLAB + chix * CH, CH)],
                    )

    return k(dst_flat, ex, invden)


def _sc_messages(src_flat, dst_flat, coef, hs_flat, zacc):
    CH2 = C // 2

    @functools.partial(
        pl.kernel,
        out_type=jax.ShapeDtypeStruct((2 * H * ACC_ROWS, CH2), jnp.float32),
        mesh=_mesh(),
        compiler_params=_sc_params(),
        scratch_types=[
            pltpu.VMEM((CH5,), jnp.int32),
            pltpu.VMEM((CH5,), jnp.int32),
            pltpu.VMEM((CH5,), jnp.int32),
            pltpu.VMEM((CH5,), jnp.float32),
            pltpu.VMEM((CH5, CH2), jnp.float32),
            pltpu.VMEM_SHARED((ACC_ROWS, CH2), jnp.float32),
        ],
    )
    def k(src_hbm, dst_hbm, coef_hbm, hs_hbm, zacc_hbm, num_hbm,
          srcv, dstv, hsix, coefv, hrows, acc):
        core = lax.axis_index("c")
        sub = lax.axis_index("s")
        w = core * NS + sub

        @pl.loop(0, 4)
        def _(pi):
            h = core * 2 + (pi >> 1)
            half = pi & 1
            p = core * 4 + pi
            pltpu.sync_copy(zacc_hbm, acc.at[pl.ds(sub * ACC_SLAB, ACC_SLAB)])
            plsc.subcore_barrier()

            @pl.loop(0, NREL)
            def _(r):
                @pl.loop(0, SLAB2 // CH5)
                def _(chix):
                    base = r * E_PAD + sub * SLAB2 + chix * CH5
                    pltpu.sync_copy(src_hbm.at[pl.ds(base, CH5)], srcv)
                    pltpu.sync_copy(dst_hbm.at[pl.ds(base, CH5)], dstv)
                    cbase = (r * H + h) * E_PAD + sub * SLAB2 + chix * CH5
                    pltpu.sync_copy(coef_hbm.at[pl.ds(cbase, CH5)], coefv)

                    hs0 = r * N_SRC * 2 * H + h * 2 + half

                    @pl.loop(0, CH5, step=16)
                    def _(i):
                        hsix[pl.ds(i, 16)] = srcv[pl.ds(i, 16)] * (2 * H) + hs0

                    pltpu.sync_copy(hs_hbm.at[hsix], hrows)

                    @pl.loop(0, CH5, step=16)
                    def _(i):
                        c16 = coefv[pl.ds(i, 16)]
                        for j in range(16):
                            cv = c16[j]
                            hrows[i + j, :] = hrows[i + j, :] * cv

                    pltpu.sync_copy(hrows, acc.at[dstv], add=True)

            plsc.subcore_barrier()
            pltpu.sync_copy(
                acc.at[pl.ds(sub * ACC_SLAB, ACC_SLAB)],
                num_hbm.at[pl.ds(p * ACC_ROWS + sub * ACC_SLAB, ACC_SLAB)],
            )
            plsc.subcore_barrier()

    return k(src_flat, dst_flat, coef, hs_flat, zacc)


def _tc_epilogue(parts, x_job, bias, ln_gamma, ln_beta):
    BLK = 1000

    W16 = C // 2

    def body(*refs):
        nrefs = refs[:8]
        xr, br, gr, btr, outr = refs[8:]
        bsum = jnp.sum(br[...], axis=0, keepdims=True)
        phs = []
        for q, nr in enumerate(nrefs):
            ph = (nr[...] + xr[:, q * W16:(q + 1) * W16]
                  + bsum[:, q * W16:(q + 1) * W16])
            phs.append(jnp.maximum(ph, 0.0))
        s1 = phs[0].sum(-1, keepdims=True)
        for p in phs[1:]:
            s1 = s1 + p.sum(-1, keepdims=True)
        mu = s1 * (1.0 / D)
        s2 = ((phs[0] - mu) ** 2).sum(-1, keepdims=True)
        for p in phs[1:]:
            s2 = s2 + ((p - mu) ** 2).sum(-1, keepdims=True)
        rstd = lax.rsqrt(s2 * (1.0 / D) + 1e-5)
        for q, p in enumerate(phs):
            outr[:, q * W16:(q + 1) * W16] = (
                (p - mu) * rstd * gr[:, q * W16:(q + 1) * W16]
                + btr[:, q * W16:(q + 1) * W16]
            )

    return pl.pallas_call(
        body,
        grid=(N_JOB // BLK,),
        in_specs=[pl.BlockSpec((BLK, W16), lambda i: (i, 0))] * 8 + [
            pl.BlockSpec((BLK, D), lambda i: (i, 0)),
            pl.BlockSpec((NREL, D), lambda i: (0, 0)),
            pl.BlockSpec((1, D), lambda i: (0, 0)),
            pl.BlockSpec((1, D), lambda i: (0, 0)),
        ],
        out_specs=pl.BlockSpec((BLK, D), lambda i: (i, 0)),
        out_shape=jax.ShapeDtypeStruct((N_JOB, D), jnp.float32),
    )(*parts, x_job, bias, ln_gamma.reshape(1, D), ln_beta.reshape(1, D))


def kernel(x_job, x_station, x_machine, x_robot, ei_can_load, ei_loaded,
           ei_will_execute, ei_execute, ei_hold, W, att_src, att_dst, bias,
           ln_gamma, ln_beta):
    eis = [ei_can_load, ei_loaded, ei_will_execute, ei_execute, ei_hold]
    xs = jnp.stack([x_station, x_station, x_machine, x_machine, x_robot])

    srcs = [jnp.pad(ei[0].astype(jnp.int32), (0, E_PAD - E)) for ei in eis]
    dsts = [jnp.pad(ei[1].astype(jnp.int32), (0, E_PAD - E)) for ei in eis]
    src_flat = jnp.concatenate(srcs)
    dst_flat = jnp.concatenate(dsts)

    eye = jnp.eye(H, dtype=jnp.float32)
    As = (att_src[:, :, :, None] * eye[:, None, :]).reshape(NREL, D, H)
    Ad4 = (att_dst[:, :, :, None] * eye[:, None, :]).reshape(NREL, D, H)
    Ad = jnp.concatenate([Ad4, jnp.zeros((NREL, D, 12), jnp.float32)], axis=-1)

    hs, asv = _tc_project(xs, W, As)
    adv = _tc_dst_alpha(x_job, W, Ad)

    zden = jnp.zeros((DEN_PSLAB, 8), jnp.float32)
    ex, den2 = _sc_edge_ex_den(
        src_flat, dst_flat, asv, adv.reshape(NREL * N_JOB, 16), zden)
    invden = _tc_invden(den2)
    coef = _sc_coef(dst_flat, ex, invden)

    zacc = jnp.zeros((ACC_SLAB, C // 2), jnp.float32)
    num = _sc_messages(
        src_flat, dst_flat, coef, hs.reshape(NREL * N_SRC * 2 * H, C // 2),
        zacc)

    parts = [lax.slice(num, (p * ACC_ROWS, 0), (p * ACC_ROWS + N_JOB, C // 2))
             for p in range(2 * H)]
    return _tc_epilogue(parts, x_job, bias, ln_gamma, ln_beta)


# speedup vs baseline: 31.7608x; 1.2093x over previous
import dataclasses
import functools

import jax
import jax.numpy as jnp
from jax import lax
from jax.experimental import pallas as pl
from jax.experimental.pallas import tpu as pltpu
from jax.experimental.pallas import tpu_sc as plsc

N_JOB = 50000
N_SRC = 10000
E = 120000
D = 128
H = 4
C = 32
NREL = 5

NC = 2
NS = 16
TILES = NC * NS

E_PAD = 122880
SLAB = E_PAD // TILES
CH = 512
CH5 = 1280

DEN_R = 250240
DEN_PR = 51200
DEN_PSLAB = DEN_PR // NS
SLAB2 = E_PAD // NS
ACC_ROWS = 50048
ACC_SLAB = ACC_ROWS // NS


def _mesh():
    return plsc.VectorSubcoreMesh(core_axis_name="c", subcore_axis_name="s")


def _sc_params():
    cp = pltpu.CompilerParams()
    if "needs_layout_passes" in pltpu.CompilerParams.__dataclass_fields__:
        cp = dataclasses.replace(cp, needs_layout_passes=False)
    if "use_tc_tiling_on_sc" in pltpu.CompilerParams.__dataclass_fields__:
        cp = dataclasses.replace(cp, use_tc_tiling_on_sc=False)
    return cp


def _tc_project(xs, W, As):
    BLK = 2000

    def body(x_ref, w_ref, a_ref, hs_ref, as_ref):
        h = jnp.dot(x_ref[0], w_ref[0], preferred_element_type=jnp.float32)
        hs_ref[0] = h
        as_ref[0] = jnp.dot(h, a_ref[0], preferred_element_type=jnp.float32)

    return pl.pallas_call(
        body,
        grid=(NREL, N_SRC // BLK),
        in_specs=[
            pl.BlockSpec((1, BLK, D), lambda r, i: (r, i, 0)),
            pl.BlockSpec((1, D, D), lambda r, i: (r, 0, 0)),
            pl.BlockSpec((1, D, H), lambda r, i: (r, 0, 0)),
        ],
        out_specs=[
            pl.BlockSpec((1, BLK, D), lambda r, i: (r, i, 0)),
            pl.BlockSpec((1, BLK, H), lambda r, i: (r, i, 0)),
        ],
        out_shape=[
            jax.ShapeDtypeStruct((NREL, N_SRC, D), jnp.float32),
            jax.ShapeDtypeStruct((NREL, N_SRC, H), jnp.float32),
        ],
    )(xs, W, As)


def _tc_dst_alpha(x_job, W, Ad):
    BLK = 2000

    def body(x_ref, w_ref, ad_ref, out_ref):
        cols = [
            jnp.dot(w_ref[r], ad_ref[r], preferred_element_type=jnp.float32)
            for r in range(NREL)
        ]
        wd = jnp.concatenate(cols, axis=1)
        out_ref[...] = jnp.dot(x_ref[...], wd, preferred_element_type=jnp.float32)

    return pl.pallas_call(
        body,
        grid=(N_JOB // BLK,),
        in_specs=[
            pl.BlockSpec((BLK, D), lambda i: (i, 0)),
            pl.BlockSpec((NREL, D, D), lambda i: (0, 0, 0)),
            pl.BlockSpec((NREL, D, 16), lambda i: (0, 0, 0)),
        ],
        out_specs=pl.BlockSpec((BLK, 16 * NREL), lambda i: (i, 0)),
        out_shape=jax.ShapeDtypeStruct((N_JOB, 16 * NREL), jnp.float32),
    )(x_job, W, Ad)


def _sc_edge_ex_den(src_flat, dst_flat, asv, adv_flat, zden):
    @functools.partial(
        pl.kernel,
        out_type=[
            jax.ShapeDtypeStruct((NREL * E_PAD, H), jnp.float32),
            jax.ShapeDtypeStruct((NREL * DEN_PR, 8), jnp.float32),
            jax.ShapeDtypeStruct((NREL * H * E_PAD,), jnp.float32),
        ],
        mesh=_mesh(),
        compiler_params=_sc_params(),
        scratch_types=[
            pltpu.VMEM((N_SRC, H), jnp.float32),
            pltpu.VMEM((CH,), jnp.int32),
            pltpu.VMEM((CH,), jnp.int32),
            pltpu.VMEM((CH,), jnp.int32),
            pltpu.VMEM((CH, 16), jnp.float32),
            pltpu.VMEM((CH, H), jnp.float32),
            pltpu.VMEM((CH, 8), jnp.float32),
            pltpu.VMEM((CH, 8), jnp.float32),
            pltpu.VMEM((H, CH), jnp.float32),
            pltpu.VMEM_SHARED((DEN_PR, 8), jnp.float32),
        ],
    )
    def k(src_hbm, dst_hbm, as_hbm, ad_hbm, zden_hbm, ex_hbm, den_hbm,
          coef_hbm, as_tab, srcv, dstv, adix, adrows, exb, exb8, denrows,
          cbuf, den_sp):
        core = lax.axis_index("c")
        sub = lax.axis_index("s")
        iota16 = lax.iota(jnp.int32, 16)

        rlo = core * 2
        rhi = 2 + core * 3

        pltpu.sync_copy(zden_hbm.at[pl.ds(0, CH)], exb8)

        @pl.loop(rlo, rhi)
        def _(r):
            pltpu.sync_copy(as_hbm.at[r], as_tab)
            pltpu.sync_copy(zden_hbm, den_sp.at[pl.ds(sub * DEN_PSLAB, DEN_PSLAB)])
            plsc.subcore_barrier()

            @pl.loop(0, SLAB2 // CH)
            def _(chix):
                base = r * E_PAD + sub * SLAB2 + chix * CH
                pltpu.sync_copy(src_hbm.at[pl.ds(base, CH)], srcv)
                pltpu.sync_copy(dst_hbm.at[pl.ds(base, CH)], dstv)

                @pl.loop(0, CH, step=16)
                def _(i):
                    adix[pl.ds(i, 16)] = dstv[pl.ds(i, 16)] * 5 + r

                pltpu.sync_copy(ad_hbm.at[adix], adrows)

                epos0 = sub * SLAB2 + chix * CH

                @pl.loop(0, CH, step=16)
                def _(i):
                    s16 = srcv[pl.ds(i, 16)]
                    pos = iota16 + (epos0 + i)
                    mask = pos < E
                    off = iota16 + i
                    for h in range(H):
                        hvec = jnp.full((16,), h, jnp.int32)
                        a_s = plsc.load_gather(as_tab, [s16, hvec])
                        a_d = plsc.load_gather(adrows, [off, hvec])
                        s = a_s + a_d
                        s = jnp.maximum(s, 0.2 * s)
                        ex = jnp.where(mask, jnp.exp(s), 0.0)
                        plsc.store_scatter(exb, [off, hvec], ex)
                        plsc.store_scatter(exb8, [off, hvec], ex)

                pltpu.sync_copy(exb, ex_hbm.at[pl.ds(base, CH)])
                pltpu.sync_copy(exb8, den_sp.at[dstv], add=True)

            plsc.subcore_barrier()
            pltpu.sync_copy(
                den_sp.at[pl.ds(sub * DEN_PSLAB, DEN_PSLAB)],
                den_hbm.at[pl.ds(r * DEN_PR + sub * DEN_PSLAB, DEN_PSLAB)],
            )
            plsc.subcore_barrier()

            @pl.loop(0, SLAB2 // CH)
            def _(chix):
                base = r * E_PAD + sub * SLAB2 + chix * CH
                pltpu.sync_copy(dst_hbm.at[pl.ds(base, CH)], dstv)
                pltpu.sync_copy(ex_hbm.at[pl.ds(base, CH)], exb)

                @pl.loop(0, CH, step=16)
                def _(i):
                    adix[pl.ds(i, 16)] = dstv[pl.ds(i, 16)] + r * DEN_PR

                pltpu.sync_copy(den_hbm.at[adix], denrows)

                @pl.loop(0, CH, step=16)
                def _(i):
                    off = iota16 + i
                    for h in range(H):
                        hvec = jnp.full((16,), h, jnp.int32)
                        exv = plsc.load_gather(exb, [off, hvec])
                        dnv = plsc.load_gather(denrows, [off, hvec])
                        cbuf[h, pl.ds(i, 16)] = exv / (dnv + 1e-16)

                for h in range(H):
                    pltpu.sync_copy(
                        cbuf.at[h],
                        coef_hbm.at[pl.ds(
                            (r * H + h) * E_PAD + sub * SLAB2 + chix * CH, CH)],
                    )

    return k(src_flat, dst_flat, asv, adv_flat, zden)


def _sc_messages(src_flat, dst_flat, coef, hs_flat, zacc):
    CH2 = C // 2

    @functools.partial(
        pl.kernel,
        out_type=jax.ShapeDtypeStruct((2 * H * ACC_ROWS, CH2), jnp.float32),
        mesh=_mesh(),
        compiler_params=_sc_params(),
        scratch_types=[
            pltpu.VMEM((CH5,), jnp.int32),
            pltpu.VMEM((CH5,), jnp.int32),
            pltpu.VMEM((CH5,), jnp.int32),
            pltpu.VMEM((CH5,), jnp.float32),
            pltpu.VMEM((CH5, CH2), jnp.float32),
            pltpu.VMEM_SHARED((ACC_ROWS, CH2), jnp.float32),
        ],
    )
    def k(src_hbm, dst_hbm, coef_hbm, hs_hbm, zacc_hbm, num_hbm,
          srcv, dstv, hsix, coefv, hrows, acc):
        core = lax.axis_index("c")
        sub = lax.axis_index("s")
        w = core * NS + sub

        @pl.loop(0, 4)
        def _(pi):
            h = core * 2 + (pi >> 1)
            half = pi & 1
            p = core * 4 + pi
            pltpu.sync_copy(zacc_hbm, acc.at[pl.ds(sub * ACC_SLAB, ACC_SLAB)])
            plsc.subcore_barrier()

            @pl.loop(0, NREL)
            def _(r):
                @pl.loop(0, SLAB2 // CH5)
                def _(chix):
                    base = r * E_PAD + sub * SLAB2 + chix * CH5
                    pltpu.sync_copy(src_hbm.at[pl.ds(base, CH5)], srcv)
                    pltpu.sync_copy(dst_hbm.at[pl.ds(base, CH5)], dstv)
                    cbase = (r * H + h) * E_PAD + sub * SLAB2 + chix * CH5
                    pltpu.sync_copy(coef_hbm.at[pl.ds(cbase, CH5)], coefv)

                    hs0 = r * N_SRC * 2 * H + h * 2 + half

                    @pl.loop(0, CH5, step=16)
                    def _(i):
                        hsix[pl.ds(i, 16)] = srcv[pl.ds(i, 16)] * (2 * H) + hs0

                    pltpu.sync_copy(hs_hbm.at[hsix], hrows)

                    @pl.loop(0, CH5, step=16)
                    def _(i):
                        c16 = coefv[pl.ds(i, 16)]
                        for j in range(16):
                            cv = c16[j]
                            hrows[i + j, :] = hrows[i + j, :] * cv

                    pltpu.sync_copy(hrows, acc.at[dstv], add=True)

            plsc.subcore_barrier()
            pltpu.sync_copy(
                acc.at[pl.ds(sub * ACC_SLAB, ACC_SLAB)],
                num_hbm.at[pl.ds(p * ACC_ROWS + sub * ACC_SLAB, ACC_SLAB)],
            )
            plsc.subcore_barrier()

    return k(src_flat, dst_flat, coef, hs_flat, zacc)


def _tc_epilogue(parts, x_job, bias, ln_gamma, ln_beta):
    BLK = 1000

    W16 = C // 2

    def body(*refs):
        nrefs = refs[:8]
        xr, br, gr, btr, outr = refs[8:]
        bsum = jnp.sum(br[...], axis=0, keepdims=True)
        phs = []
        for q, nr in enumerate(nrefs):
            ph = (nr[...] + xr[:, q * W16:(q + 1) * W16]
                  + bsum[:, q * W16:(q + 1) * W16])
            phs.append(jnp.maximum(ph, 0.0))
        s1 = phs[0].sum(-1, keepdims=True)
        for p in phs[1:]:
            s1 = s1 + p.sum(-1, keepdims=True)
        mu = s1 * (1.0 / D)
        s2 = ((phs[0] - mu) ** 2).sum(-1, keepdims=True)
        for p in phs[1:]:
            s2 = s2 + ((p - mu) ** 2).sum(-1, keepdims=True)
        rstd = lax.rsqrt(s2 * (1.0 / D) + 1e-5)
        for q, p in enumerate(phs):
            outr[:, q * W16:(q + 1) * W16] = (
                (p - mu) * rstd * gr[:, q * W16:(q + 1) * W16]
                + btr[:, q * W16:(q + 1) * W16]
            )

    return pl.pallas_call(
        body,
        grid=(N_JOB // BLK,),
        in_specs=[pl.BlockSpec((BLK, W16), lambda i: (i, 0))] * 8 + [
            pl.BlockSpec((BLK, D), lambda i: (i, 0)),
            pl.BlockSpec((NREL, D), lambda i: (0, 0)),
            pl.BlockSpec((1, D), lambda i: (0, 0)),
            pl.BlockSpec((1, D), lambda i: (0, 0)),
        ],
        out_specs=pl.BlockSpec((BLK, D), lambda i: (i, 0)),
        out_shape=jax.ShapeDtypeStruct((N_JOB, D), jnp.float32),
    )(*parts, x_job, bias, ln_gamma.reshape(1, D), ln_beta.reshape(1, D))


def kernel(x_job, x_station, x_machine, x_robot, ei_can_load, ei_loaded,
           ei_will_execute, ei_execute, ei_hold, W, att_src, att_dst, bias,
           ln_gamma, ln_beta):
    eis = [ei_can_load, ei_loaded, ei_will_execute, ei_execute, ei_hold]
    xs = jnp.stack([x_station, x_station, x_machine, x_machine, x_robot])

    srcs = [jnp.pad(ei[0].astype(jnp.int32), (0, E_PAD - E)) for ei in eis]
    dsts = [jnp.pad(ei[1].astype(jnp.int32), (0, E_PAD - E)) for ei in eis]
    src_flat = jnp.concatenate(srcs)
    dst_flat = jnp.concatenate(dsts)

    eye = jnp.eye(H, dtype=jnp.float32)
    As = (att_src[:, :, :, None] * eye[:, None, :]).reshape(NREL, D, H)
    Ad4 = (att_dst[:, :, :, None] * eye[:, None, :]).reshape(NREL, D, H)
    Ad = jnp.concatenate([Ad4, jnp.zeros((NREL, D, 12), jnp.float32)], axis=-1)

    hs, asv = _tc_project(xs, W, As)
    adv = _tc_dst_alpha(x_job, W, Ad)

    zden = jnp.zeros((DEN_PSLAB, 8), jnp.float32)
    ex, den, coef = _sc_edge_ex_den(
        src_flat, dst_flat, asv, adv.reshape(NREL * N_JOB, 16), zden)

    zacc = jnp.zeros((ACC_SLAB, C // 2), jnp.float32)
    num = _sc_messages(
        src_flat, dst_flat, coef, hs.reshape(NREL * N_SRC * 2 * H, C // 2),
        zacc)

    parts = [lax.slice(num, (p * ACC_ROWS, 0), (p * ACC_ROWS + N_JOB, C // 2))
             for p in range(2 * H)]
    return _tc_epilogue(parts, x_job, bias, ln_gamma, ln_beta)


# speedup vs baseline: 33.5415x; 1.0561x over previous
import dataclasses
import functools

import jax
import jax.numpy as jnp
from jax import lax
from jax.experimental import pallas as pl
from jax.experimental.pallas import tpu as pltpu
from jax.experimental.pallas import tpu_sc as plsc

N_JOB = 50000
N_SRC = 10000
E = 120000
D = 128
H = 4
C = 32
NREL = 5

NC = 2
NS = 16
TILES = NC * NS

E_PAD = 122880
SLAB = E_PAD // TILES
CH = 512
CH5 = 1280

DEN_R = 250240
DEN_PR = 51200
DEN_PSLAB = DEN_PR // NS
SLAB2 = E_PAD // NS
ACC_ROWS = 50048
ACC_SLAB = ACC_ROWS // NS


def _mesh():
    return plsc.VectorSubcoreMesh(core_axis_name="c", subcore_axis_name="s")


def _sc_params():
    cp = pltpu.CompilerParams()
    if "needs_layout_passes" in pltpu.CompilerParams.__dataclass_fields__:
        cp = dataclasses.replace(cp, needs_layout_passes=False)
    if "use_tc_tiling_on_sc" in pltpu.CompilerParams.__dataclass_fields__:
        cp = dataclasses.replace(cp, use_tc_tiling_on_sc=False)
    return cp


def _tc_project(xs, W, As):
    BLK = 2000

    def body(x_ref, w_ref, a_ref, hs_ref, as_ref):
        h = jnp.dot(x_ref[0], w_ref[0], preferred_element_type=jnp.float32)
        hs_ref[0] = h
        as_ref[0] = jnp.dot(h, a_ref[0], preferred_element_type=jnp.float32)

    return pl.pallas_call(
        body,
        grid=(NREL, N_SRC // BLK),
        in_specs=[
            pl.BlockSpec((1, BLK, D), lambda r, i: (r, i, 0)),
            pl.BlockSpec((1, D, D), lambda r, i: (r, 0, 0)),
            pl.BlockSpec((1, D, H), lambda r, i: (r, 0, 0)),
        ],
        out_specs=[
            pl.BlockSpec((1, BLK, D), lambda r, i: (r, i, 0)),
            pl.BlockSpec((1, BLK, H), lambda r, i: (r, i, 0)),
        ],
        out_shape=[
            jax.ShapeDtypeStruct((NREL, N_SRC, D), jnp.float32),
            jax.ShapeDtypeStruct((NREL, N_SRC, H), jnp.float32),
        ],
    )(xs, W, As)


def _tc_dst_alpha(x_job, W, Ad):
    BLK = 2000

    def body(x_ref, w_ref, ad_ref, out_ref):
        cols = [
            jnp.dot(w_ref[r], ad_ref[r], preferred_element_type=jnp.float32)
            for r in range(NREL)
        ]
        wd = jnp.concatenate(cols, axis=1)
        out_ref[...] = jnp.dot(x_ref[...], wd, preferred_element_type=jnp.float32)

    return pl.pallas_call(
        body,
        grid=(N_JOB // BLK,),
        in_specs=[
            pl.BlockSpec((BLK, D), lambda i: (i, 0)),
            pl.BlockSpec((NREL, D, D), lambda i: (0, 0, 0)),
            pl.BlockSpec((NREL, D, 16), lambda i: (0, 0, 0)),
        ],
        out_specs=pl.BlockSpec((BLK, 16 * NREL), lambda i: (i, 0)),
        out_shape=jax.ShapeDtypeStruct((N_JOB, 16 * NREL), jnp.float32),
    )(x_job, W, Ad)


def _sc_edge_ex_den(src_flat, dst_flat, asv, adv_flat, zden):
    @functools.partial(
        pl.kernel,
        out_type=[
            jax.ShapeDtypeStruct((NREL * E_PAD, H), jnp.float32),
            jax.ShapeDtypeStruct((NREL * DEN_PR, 8), jnp.float32),
            jax.ShapeDtypeStruct((NREL * H * E_PAD,), jnp.float32),
        ],
        mesh=_mesh(),
        compiler_params=_sc_params(),
        scratch_types=[
            pltpu.VMEM((N_SRC, H), jnp.float32),
            pltpu.VMEM((CH,), jnp.int32),
            pltpu.VMEM((CH,), jnp.int32),
            pltpu.VMEM((CH,), jnp.int32),
            pltpu.VMEM((CH, 16), jnp.float32),
            pltpu.VMEM((CH, H), jnp.float32),
            pltpu.VMEM((CH, 8), jnp.float32),
            pltpu.VMEM((CH, 8), jnp.float32),
            pltpu.VMEM((H, CH), jnp.float32),
            pltpu.VMEM_SHARED((DEN_PR, 8), jnp.float32),
        ],
    )
    def k(src_hbm, dst_hbm, as_hbm, ad_hbm, zden_hbm, ex_hbm, den_hbm,
          coef_hbm, as_tab, srcv, dstv, adix, adrows, exb, exb8, denrows,
          cbuf, den_sp):
        core = lax.axis_index("c")
        sub = lax.axis_index("s")
        iota16 = lax.iota(jnp.int32, 16)

        rlo = core * 2
        rhi = 2 + core * 3

        pltpu.sync_copy(zden_hbm.at[pl.ds(0, CH)], exb8)

        @pl.loop(rlo, rhi)
        def _(r):
            pltpu.sync_copy(as_hbm.at[r], as_tab)
            pltpu.sync_copy(zden_hbm, den_sp.at[pl.ds(sub * DEN_PSLAB, DEN_PSLAB)])
            plsc.subcore_barrier()

            @pl.loop(0, SLAB2 // CH)
            def _(chix):
                base = r * E_PAD + sub * SLAB2 + chix * CH
                pltpu.sync_copy(
                    (src_hbm.at[pl.ds(base, CH)], dst_hbm.at[pl.ds(base, CH)]),
                    (srcv, dstv))

                @pl.loop(0, CH, step=16)
                def _(i):
                    adix[pl.ds(i, 16)] = dstv[pl.ds(i, 16)] * 5 + r

                pltpu.sync_copy(ad_hbm.at[adix], adrows)

                epos0 = sub * SLAB2 + chix * CH

                @pl.loop(0, CH, step=16)
                def _(i):
                    s16 = srcv[pl.ds(i, 16)]
                    pos = iota16 + (epos0 + i)
                    mask = pos < E
                    off = iota16 + i
                    for h in range(H):
                        hvec = jnp.full((16,), h, jnp.int32)
                        a_s = plsc.load_gather(as_tab, [s16, hvec])
                        a_d = plsc.load_gather(adrows, [off, hvec])
                        s = a_s + a_d
                        s = jnp.maximum(s, 0.2 * s)
                        ex = jnp.where(mask, jnp.exp(s), 0.0)
                        plsc.store_scatter(exb, [off, hvec], ex)
                        plsc.store_scatter(exb8, [off, hvec], ex)

                pltpu.sync_copy(exb, ex_hbm.at[pl.ds(base, CH)])
                pltpu.sync_copy(exb8, den_sp.at[dstv], add=True)

            plsc.subcore_barrier()
            pltpu.sync_copy(
                den_sp.at[pl.ds(sub * DEN_PSLAB, DEN_PSLAB)],
                den_hbm.at[pl.ds(r * DEN_PR + sub * DEN_PSLAB, DEN_PSLAB)],
            )
            plsc.subcore_barrier()

            @pl.loop(0, SLAB2 // CH)
            def _(chix):
                base = r * E_PAD + sub * SLAB2 + chix * CH
                pltpu.sync_copy(
                    (dst_hbm.at[pl.ds(base, CH)], ex_hbm.at[pl.ds(base, CH)]),
                    (dstv, exb))

                @pl.loop(0, CH, step=16)
                def _(i):
                    adix[pl.ds(i, 16)] = dstv[pl.ds(i, 16)] + r * DEN_PR

                pltpu.sync_copy(den_hbm.at[adix], denrows)

                @pl.loop(0, CH, step=16)
                def _(i):
                    off = iota16 + i
                    for h in range(H):
                        hvec = jnp.full((16,), h, jnp.int32)
                        exv = plsc.load_gather(exb, [off, hvec])
                        dnv = plsc.load_gather(denrows, [off, hvec])
                        cbuf[h, pl.ds(i, 16)] = exv / (dnv + 1e-16)

                for h in range(H):
                    pltpu.sync_copy(
                        cbuf.at[h],
                        coef_hbm.at[pl.ds(
                            (r * H + h) * E_PAD + sub * SLAB2 + chix * CH, CH)],
                    )

    return k(src_flat, dst_flat, asv, adv_flat, zden)


def _sc_messages(src_flat, dst_flat, coef, hs_flat, zacc):
    CH2 = C // 2

    @functools.partial(
        pl.kernel,
        out_type=jax.ShapeDtypeStruct((2 * H * ACC_ROWS, CH2), jnp.float32),
        mesh=_mesh(),
        compiler_params=_sc_params(),
        scratch_types=[
            pltpu.VMEM((CH5,), jnp.int32),
            pltpu.VMEM((CH5,), jnp.int32),
            pltpu.VMEM((CH5,), jnp.int32),
            pltpu.VMEM((CH5,), jnp.float32),
            pltpu.VMEM((CH5, CH2), jnp.float32),
            pltpu.VMEM_SHARED((ACC_ROWS, CH2), jnp.float32),
        ],
    )
    def k(src_hbm, dst_hbm, coef_hbm, hs_hbm, zacc_hbm, num_hbm,
          srcv, dstv, hsix, coefv, hrows, acc):
        core = lax.axis_index("c")
        sub = lax.axis_index("s")
        w = core * NS + sub

        @pl.loop(0, 4)
        def _(pi):
            h = core * 2 + (pi >> 1)
            half = pi & 1
            p = core * 4 + pi
            pltpu.sync_copy(zacc_hbm, acc.at[pl.ds(sub * ACC_SLAB, ACC_SLAB)])
            plsc.subcore_barrier()

            @pl.loop(0, NREL)
            def _(r):
                @pl.loop(0, SLAB2 // CH5)
                def _(chix):
                    base = r * E_PAD + sub * SLAB2 + chix * CH5
                    cbase = (r * H + h) * E_PAD + sub * SLAB2 + chix * CH5
                    pltpu.sync_copy(
                        (src_hbm.at[pl.ds(base, CH5)],
                         dst_hbm.at[pl.ds(base, CH5)],
                         coef_hbm.at[pl.ds(cbase, CH5)]),
                        (srcv, dstv, coefv))

                    hs0 = r * N_SRC * 2 * H + h * 2 + half

                    @pl.loop(0, CH5, step=16)
                    def _(i):
                        hsix[pl.ds(i, 16)] = srcv[pl.ds(i, 16)] * (2 * H) + hs0

                    pltpu.sync_copy(hs_hbm.at[hsix], hrows)

                    @pl.loop(0, CH5, step=16)
                    def _(i):
                        c16 = coefv[pl.ds(i, 16)]
                        for j in range(16):
                            cv = c16[j]
                            hrows[i + j, :] = hrows[i + j, :] * cv

                    pltpu.sync_copy(hrows, acc.at[dstv], add=True)

            plsc.subcore_barrier()
            pltpu.sync_copy(
                acc.at[pl.ds(sub * ACC_SLAB, ACC_SLAB)],
                num_hbm.at[pl.ds(p * ACC_ROWS + sub * ACC_SLAB, ACC_SLAB)],
            )
            plsc.subcore_barrier()

    return k(src_flat, dst_flat, coef, hs_flat, zacc)


def _tc_epilogue(parts, x_job, bias, ln_gamma, ln_beta):
    BLK = 1000

    W16 = C // 2

    def body(*refs):
        nrefs = refs[:8]
        xr, br, gr, btr, outr = refs[8:]
        bsum = jnp.sum(br[...], axis=0, keepdims=True)
        phs = []
        for q, nr in enumerate(nrefs):
            ph = (nr[...] + xr[:, q * W16:(q + 1) * W16]
                  + bsum[:, q * W16:(q + 1) * W16])
            phs.append(jnp.maximum(ph, 0.0))
        s1 = phs[0].sum(-1, keepdims=True)
        for p in phs[1:]:
            s1 = s1 + p.sum(-1, keepdims=True)
        mu = s1 * (1.0 / D)
        s2 = ((phs[0] - mu) ** 2).sum(-1, keepdims=True)
        for p in phs[1:]:
            s2 = s2 + ((p - mu) ** 2).sum(-1, keepdims=True)
        rstd = lax.rsqrt(s2 * (1.0 / D) + 1e-5)
        for q, p in enumerate(phs):
            outr[:, q * W16:(q + 1) * W16] = (
                (p - mu) * rstd * gr[:, q * W16:(q + 1) * W16]
                + btr[:, q * W16:(q + 1) * W16]
            )

    return pl.pallas_call(
        body,
        grid=(N_JOB // BLK,),
        in_specs=[pl.BlockSpec((BLK, W16), lambda i: (i, 0))] * 8 + [
            pl.BlockSpec((BLK, D), lambda i: (i, 0)),
            pl.BlockSpec((NREL, D), lambda i: (0, 0)),
            pl.BlockSpec((1, D), lambda i: (0, 0)),
            pl.BlockSpec((1, D), lambda i: (0, 0)),
        ],
        out_specs=pl.BlockSpec((BLK, D), lambda i: (i, 0)),
        out_shape=jax.ShapeDtypeStruct((N_JOB, D), jnp.float32),
    )(*parts, x_job, bias, ln_gamma.reshape(1, D), ln_beta.reshape(1, D))


def kernel(x_job, x_station, x_machine, x_robot, ei_can_load, ei_loaded,
           ei_will_execute, ei_execute, ei_hold, W, att_src, att_dst, bias,
           ln_gamma, ln_beta):
    eis = [ei_can_load, ei_loaded, ei_will_execute, ei_execute, ei_hold]
    xs = jnp.stack([x_station, x_station, x_machine, x_machine, x_robot])

    srcs = [jnp.pad(ei[0].astype(jnp.int32), (0, E_PAD - E)) for ei in eis]
    dsts = [jnp.pad(ei[1].astype(jnp.int32), (0, E_PAD - E)) for ei in eis]
    src_flat = jnp.concatenate(srcs)
    dst_flat = jnp.concatenate(dsts)

    eye = jnp.eye(H, dtype=jnp.float32)
    As = (att_src[:, :, :, None] * eye[:, None, :]).reshape(NREL, D, H)
    Ad4 = (att_dst[:, :, :, None] * eye[:, None, :]).reshape(NREL, D, H)
    Ad = jnp.concatenate([Ad4, jnp.zeros((NREL, D, 12), jnp.float32)], axis=-1)

    hs, asv = _tc_project(xs, W, As)
    adv = _tc_dst_alpha(x_job, W, Ad)

    zden = jnp.zeros((DEN_PSLAB, 8), jnp.float32)
    ex, den, coef = _sc_edge_ex_den(
        src_flat, dst_flat, asv, adv.reshape(NREL * N_JOB, 16), zden)

    zacc = jnp.zeros((ACC_SLAB, C // 2), jnp.float32)
    num = _sc_messages(
        src_flat, dst_flat, coef, hs.reshape(NREL * N_SRC * 2 * H, C // 2),
        zacc)

    parts = [lax.slice(num, (p * ACC_ROWS, 0), (p * ACC_ROWS + N_JOB, C // 2))
             for p in range(2 * H)]
    return _tc_epilogue(parts, x_job, bias, ln_gamma, ln_beta)


# speedup vs baseline: 34.7401x; 1.0357x over previous
import dataclasses
import functools

import jax
import jax.numpy as jnp
from jax import lax
from jax.experimental import pallas as pl
from jax.experimental.pallas import tpu as pltpu
from jax.experimental.pallas import tpu_sc as plsc

N_JOB = 50000
N_SRC = 10000
E = 120000
D = 128
H = 4
C = 32
NREL = 5

NC = 2
NS = 16
TILES = NC * NS

E_PAD = 122880
SLAB = E_PAD // TILES
CH = 512
CH5 = 2560

DEN_R = 250240
DEN_PR = 51200
DEN_PSLAB = DEN_PR // NS
SLAB2 = E_PAD // NS
ACC_ROWS = 50048
ACC_SLAB = ACC_ROWS // NS


def _mesh():
    return plsc.VectorSubcoreMesh(core_axis_name="c", subcore_axis_name="s")


def _sc_params():
    cp = pltpu.CompilerParams()
    if "needs_layout_passes" in pltpu.CompilerParams.__dataclass_fields__:
        cp = dataclasses.replace(cp, needs_layout_passes=False)
    if "use_tc_tiling_on_sc" in pltpu.CompilerParams.__dataclass_fields__:
        cp = dataclasses.replace(cp, use_tc_tiling_on_sc=False)
    return cp


def _tc_project(xs, W, As):
    BLK = 2000

    def body(x_ref, w_ref, a_ref, hs_ref, as_ref):
        h = jnp.dot(x_ref[0], w_ref[0], preferred_element_type=jnp.float32)
        hs_ref[0] = h
        as_ref[0] = jnp.dot(h, a_ref[0], preferred_element_type=jnp.float32)

    return pl.pallas_call(
        body,
        grid=(NREL, N_SRC // BLK),
        in_specs=[
            pl.BlockSpec((1, BLK, D), lambda r, i: (r, i, 0)),
            pl.BlockSpec((1, D, D), lambda r, i: (r, 0, 0)),
            pl.BlockSpec((1, D, H), lambda r, i: (r, 0, 0)),
        ],
        out_specs=[
            pl.BlockSpec((1, BLK, D), lambda r, i: (r, i, 0)),
            pl.BlockSpec((1, BLK, H), lambda r, i: (r, i, 0)),
        ],
        out_shape=[
            jax.ShapeDtypeStruct((NREL, N_SRC, D), jnp.float32),
            jax.ShapeDtypeStruct((NREL, N_SRC, H), jnp.float32),
        ],
    )(xs, W, As)


def _tc_dst_alpha(x_job, W, Ad):
    BLK = 2000

    def body(x_ref, w_ref, ad_ref, out_ref):
        cols = [
            jnp.dot(w_ref[r], ad_ref[r], preferred_element_type=jnp.float32)
            for r in range(NREL)
        ]
        wd = jnp.concatenate(cols, axis=1)
        out_ref[...] = jnp.dot(x_ref[...], wd, preferred_element_type=jnp.float32)

    return pl.pallas_call(
        body,
        grid=(N_JOB // BLK,),
        in_specs=[
            pl.BlockSpec((BLK, D), lambda i: (i, 0)),
            pl.BlockSpec((NREL, D, D), lambda i: (0, 0, 0)),
            pl.BlockSpec((NREL, D, 16), lambda i: (0, 0, 0)),
        ],
        out_specs=pl.BlockSpec((BLK, 16 * NREL), lambda i: (i, 0)),
        out_shape=jax.ShapeDtypeStruct((N_JOB, 16 * NREL), jnp.float32),
    )(x_job, W, Ad)


def _sc_edge_ex_den(src_flat, dst_flat, asv, adv_flat, zden):
    @functools.partial(
        pl.kernel,
        out_type=[
            jax.ShapeDtypeStruct((NREL * E_PAD, H), jnp.float32),
            jax.ShapeDtypeStruct((NREL * DEN_PR, 8), jnp.float32),
            jax.ShapeDtypeStruct((NREL * H * E_PAD,), jnp.float32),
        ],
        mesh=_mesh(),
        compiler_params=_sc_params(),
        scratch_types=[
            pltpu.VMEM((N_SRC, H), jnp.float32),
            pltpu.VMEM((CH,), jnp.int32),
            pltpu.VMEM((CH,), jnp.int32),
            pltpu.VMEM((CH,), jnp.int32),
            pltpu.VMEM((CH, 16), jnp.float32),
            pltpu.VMEM((CH, H), jnp.float32),
            pltpu.VMEM((CH, 8), jnp.float32),
            pltpu.VMEM((CH, 8), jnp.float32),
            pltpu.VMEM((H, CH), jnp.float32),
            pltpu.VMEM_SHARED((DEN_PR, 8), jnp.float32),
        ],
    )
    def k(src_hbm, dst_hbm, as_hbm, ad_hbm, zden_hbm, ex_hbm, den_hbm,
          coef_hbm, as_tab, srcv, dstv, adix, adrows, exb, exb8, denrows,
          cbuf, den_sp):
        core = lax.axis_index("c")
        sub = lax.axis_index("s")
        iota16 = lax.iota(jnp.int32, 16)

        rlo = core * 2
        rhi = 2 + core * 3

        pltpu.sync_copy(zden_hbm.at[pl.ds(0, CH)], exb8)

        @pl.loop(rlo, rhi)
        def _(r):
            pltpu.sync_copy(as_hbm.at[r], as_tab)
            pltpu.sync_copy(zden_hbm, den_sp.at[pl.ds(sub * DEN_PSLAB, DEN_PSLAB)])
            plsc.subcore_barrier()

            @pl.loop(0, SLAB2 // CH)
            def _(chix):
                base = r * E_PAD + sub * SLAB2 + chix * CH
                pltpu.sync_copy(
                    (src_hbm.at[pl.ds(base, CH)], dst_hbm.at[pl.ds(base, CH)]),
                    (srcv, dstv))

                @pl.loop(0, CH, step=16)
                def _(i):
                    adix[pl.ds(i, 16)] = dstv[pl.ds(i, 16)] * 5 + r

                pltpu.sync_copy(ad_hbm.at[adix], adrows)

                epos0 = sub * SLAB2 + chix * CH

                @pl.loop(0, CH, step=16)
                def _(i):
                    s16 = srcv[pl.ds(i, 16)]
                    pos = iota16 + (epos0 + i)
                    mask = pos < E
                    off = iota16 + i
                    for h in range(H):
                        hvec = jnp.full((16,), h, jnp.int32)
                        a_s = plsc.load_gather(as_tab, [s16, hvec])
                        a_d = plsc.load_gather(adrows, [off, hvec])
                        s = a_s + a_d
                        s = jnp.maximum(s, 0.2 * s)
                        ex = jnp.where(mask, jnp.exp(s), 0.0)
                        plsc.store_scatter(exb, [off, hvec], ex)
                        plsc.store_scatter(exb8, [off, hvec], ex)

                pltpu.sync_copy(exb, ex_hbm.at[pl.ds(base, CH)])
                pltpu.sync_copy(exb8, den_sp.at[dstv], add=True)

            plsc.subcore_barrier()
            pltpu.sync_copy(
                den_sp.at[pl.ds(sub * DEN_PSLAB, DEN_PSLAB)],
                den_hbm.at[pl.ds(r * DEN_PR + sub * DEN_PSLAB, DEN_PSLAB)],
            )
            plsc.subcore_barrier()

            @pl.loop(0, SLAB2 // CH)
            def _(chix):
                base = r * E_PAD + sub * SLAB2 + chix * CH
                pltpu.sync_copy(
                    (dst_hbm.at[pl.ds(base, CH)], ex_hbm.at[pl.ds(base, CH)]),
                    (dstv, exb))

                @pl.loop(0, CH, step=16)
                def _(i):
                    adix[pl.ds(i, 16)] = dstv[pl.ds(i, 16)] + r * DEN_PR

                pltpu.sync_copy(den_hbm.at[adix], denrows)

                @pl.loop(0, CH, step=16)
                def _(i):
                    off = iota16 + i
                    for h in range(H):
                        hvec = jnp.full((16,), h, jnp.int32)
                        exv = plsc.load_gather(exb, [off, hvec])
                        dnv = plsc.load_gather(denrows, [off, hvec])
                        cbuf[h, pl.ds(i, 16)] = exv / (dnv + 1e-16)

                for h in range(H):
                    pltpu.sync_copy(
                        cbuf.at[h],
                        coef_hbm.at[pl.ds(
                            (r * H + h) * E_PAD + sub * SLAB2 + chix * CH, CH)],
                    )

    return k(src_flat, dst_flat, asv, adv_flat, zden)


def _sc_messages(src_flat, dst_flat, coef, hs_flat, zacc):
    CH2 = C // 2

    @functools.partial(
        pl.kernel,
        out_type=jax.ShapeDtypeStruct((2 * H * ACC_ROWS, CH2), jnp.float32),
        mesh=_mesh(),
        compiler_params=_sc_params(),
        scratch_types=[
            pltpu.VMEM((CH5,), jnp.int32),
            pltpu.VMEM((CH5,), jnp.int32),
            pltpu.VMEM((CH5,), jnp.int32),
            pltpu.VMEM((CH5,), jnp.float32),
            pltpu.VMEM((CH5, CH2), jnp.float32),
            pltpu.VMEM_SHARED((ACC_ROWS, CH2), jnp.float32),
        ],
    )
    def k(src_hbm, dst_hbm, coef_hbm, hs_hbm, zacc_hbm, num_hbm,
          srcv, dstv, hsix, coefv, hrows, acc):
        core = lax.axis_index("c")
        sub = lax.axis_index("s")
        w = core * NS + sub

        @pl.loop(0, 4)
        def _(pi):
            h = core * 2 + (pi >> 1)
            half = pi & 1
            p = core * 4 + pi
            pltpu.sync_copy(zacc_hbm, acc.at[pl.ds(sub * ACC_SLAB, ACC_SLAB)])
            plsc.subcore_barrier()

            @pl.loop(0, NREL)
            def _(r):
                @pl.loop(0, SLAB2 // CH5)
                def _(chix):
                    base = r * E_PAD + sub * SLAB2 + chix * CH5
                    cbase = (r * H + h) * E_PAD + sub * SLAB2 + chix * CH5
                    pltpu.sync_copy(
                        (src_hbm.at[pl.ds(base, CH5)],
                         dst_hbm.at[pl.ds(base, CH5)],
                         coef_hbm.at[pl.ds(cbase, CH5)]),
                        (srcv, dstv, coefv))

                    hs0 = r * N_SRC * 2 * H + h * 2 + half

                    @pl.loop(0, CH5, step=16)
                    def _(i):
                        hsix[pl.ds(i, 16)] = srcv[pl.ds(i, 16)] * (2 * H) + hs0

                    pltpu.sync_copy(hs_hbm.at[hsix], hrows)

                    @pl.loop(0, CH5, step=16)
                    def _(i):
                        c16 = coefv[pl.ds(i, 16)]
                        for j in range(16):
                            cv = c16[j]
                            hrows[i + j, :] = hrows[i + j, :] * cv

                    pltpu.sync_copy(hrows, acc.at[dstv], add=True)

            plsc.subcore_barrier()
            pltpu.sync_copy(
                acc.at[pl.ds(sub * ACC_SLAB, ACC_SLAB)],
                num_hbm.at[pl.ds(p * ACC_ROWS + sub * ACC_SLAB, ACC_SLAB)],
            )
            plsc.subcore_barrier()

    return k(src_flat, dst_flat, coef, hs_flat, zacc)


def _tc_epilogue(parts, x_job, bias, ln_gamma, ln_beta):
    BLK = 1000

    W16 = C // 2

    def body(*refs):
        nrefs = refs[:8]
        xr, br, gr, btr, outr = refs[8:]
        bsum = jnp.sum(br[...], axis=0, keepdims=True)
        phs = []
        for q, nr in enumerate(nrefs):
            ph = (nr[...] + xr[:, q * W16:(q + 1) * W16]
                  + bsum[:, q * W16:(q + 1) * W16])
            phs.append(jnp.maximum(ph, 0.0))
        s1 = phs[0].sum(-1, keepdims=True)
        for p in phs[1:]:
            s1 = s1 + p.sum(-1, keepdims=True)
        mu = s1 * (1.0 / D)
        s2 = ((phs[0] - mu) ** 2).sum(-1, keepdims=True)
        for p in phs[1:]:
            s2 = s2 + ((p - mu) ** 2).sum(-1, keepdims=True)
        rstd = lax.rsqrt(s2 * (1.0 / D) + 1e-5)
        for q, p in enumerate(phs):
            outr[:, q * W16:(q + 1) * W16] = (
                (p - mu) * rstd * gr[:, q * W16:(q + 1) * W16]
                + btr[:, q * W16:(q + 1) * W16]
            )

    return pl.pallas_call(
        body,
        grid=(N_JOB // BLK,),
        in_specs=[pl.BlockSpec((BLK, W16), lambda i: (i, 0))] * 8 + [
            pl.BlockSpec((BLK, D), lambda i: (i, 0)),
            pl.BlockSpec((NREL, D), lambda i: (0, 0)),
            pl.BlockSpec((1, D), lambda i: (0, 0)),
            pl.BlockSpec((1, D), lambda i: (0, 0)),
        ],
        out_specs=pl.BlockSpec((BLK, D), lambda i: (i, 0)),
        out_shape=jax.ShapeDtypeStruct((N_JOB, D), jnp.float32),
    )(*parts, x_job, bias, ln_gamma.reshape(1, D), ln_beta.reshape(1, D))


def kernel(x_job, x_station, x_machine, x_robot, ei_can_load, ei_loaded,
           ei_will_execute, ei_execute, ei_hold, W, att_src, att_dst, bias,
           ln_gamma, ln_beta):
    eis = [ei_can_load, ei_loaded, ei_will_execute, ei_execute, ei_hold]
    xs = jnp.stack([x_station, x_station, x_machine, x_machine, x_robot])

    srcs = [jnp.pad(ei[0].astype(jnp.int32), (0, E_PAD - E)) for ei in eis]
    dsts = [jnp.pad(ei[1].astype(jnp.int32), (0, E_PAD - E)) for ei in eis]
    src_flat = jnp.concatenate(srcs)
    dst_flat = jnp.concatenate(dsts)

    eye = jnp.eye(H, dtype=jnp.float32)
    As = (att_src[:, :, :, None] * eye[:, None, :]).reshape(NREL, D, H)
    Ad4 = (att_dst[:, :, :, None] * eye[:, None, :]).reshape(NREL, D, H)
    Ad = jnp.concatenate([Ad4, jnp.zeros((NREL, D, 12), jnp.float32)], axis=-1)

    hs, asv = _tc_project(xs, W, As)
    adv = _tc_dst_alpha(x_job, W, Ad)

    zden = jnp.zeros((DEN_PSLAB, 8), jnp.float32)
    ex, den, coef = _sc_edge_ex_den(
        src_flat, dst_flat, asv, adv.reshape(NREL * N_JOB, 16), zden)

    zacc = jnp.zeros((ACC_SLAB, C // 2), jnp.float32)
    num = _sc_messages(
        src_flat, dst_flat, coef, hs.reshape(NREL * N_SRC * 2 * H, C // 2),
        zacc)

    parts = [lax.slice(num, (p * ACC_ROWS, 0), (p * ACC_ROWS + N_JOB, C // 2))
             for p in range(2 * H)]
    return _tc_epilogue(parts, x_job, bias, ln_gamma, ln_beta)


# speedup vs baseline: 35.3410x; 1.0173x over previous
import dataclasses
import functools

import jax
import jax.numpy as jnp
from jax import lax
from jax.experimental import pallas as pl
from jax.experimental.pallas import tpu as pltpu
from jax.experimental.pallas import tpu_sc as plsc

N_JOB = 50000
N_SRC = 10000
E = 120000
D = 128
H = 4
C = 32
NREL = 5

NC = 2
NS = 16
TILES = NC * NS

E_PAD = 122880
SLAB = E_PAD // TILES
CH = 512
CH5 = 3840

DEN_R = 250240
DEN_PR = 51200
DEN_PSLAB = DEN_PR // NS
SLAB2 = E_PAD // NS
ACC_ROWS = 50048
ACC_SLAB = ACC_ROWS // NS


def _mesh():
    return plsc.VectorSubcoreMesh(core_axis_name="c", subcore_axis_name="s")


def _sc_params():
    cp = pltpu.CompilerParams()
    if "needs_layout_passes" in pltpu.CompilerParams.__dataclass_fields__:
        cp = dataclasses.replace(cp, needs_layout_passes=False)
    if "use_tc_tiling_on_sc" in pltpu.CompilerParams.__dataclass_fields__:
        cp = dataclasses.replace(cp, use_tc_tiling_on_sc=False)
    return cp


def _tc_project(xs, W, As):
    BLK = 2000

    def body(x_ref, w_ref, a_ref, hs_ref, as_ref):
        h = jnp.dot(x_ref[0], w_ref[0], preferred_element_type=jnp.float32)
        hs_ref[0] = h
        as_ref[0] = jnp.dot(h, a_ref[0], preferred_element_type=jnp.float32)

    return pl.pallas_call(
        body,
        grid=(NREL, N_SRC // BLK),
        in_specs=[
            pl.BlockSpec((1, BLK, D), lambda r, i: (r, i, 0)),
            pl.BlockSpec((1, D, D), lambda r, i: (r, 0, 0)),
            pl.BlockSpec((1, D, H), lambda r, i: (r, 0, 0)),
        ],
        out_specs=[
            pl.BlockSpec((1, BLK, D), lambda r, i: (r, i, 0)),
            pl.BlockSpec((1, BLK, H), lambda r, i: (r, i, 0)),
        ],
        out_shape=[
            jax.ShapeDtypeStruct((NREL, N_SRC, D), jnp.float32),
            jax.ShapeDtypeStruct((NREL, N_SRC, H), jnp.float32),
        ],
    )(xs, W, As)


def _tc_dst_alpha(x_job, W, Ad):
    BLK = 2000

    def body(x_ref, w_ref, ad_ref, out_ref):
        cols = [
            jnp.dot(w_ref[r], ad_ref[r], preferred_element_type=jnp.float32)
            for r in range(NREL)
        ]
        wd = jnp.concatenate(cols, axis=1)
        out_ref[...] = jnp.dot(x_ref[...], wd, preferred_element_type=jnp.float32)

    return pl.pallas_call(
        body,
        grid=(N_JOB // BLK,),
        in_specs=[
            pl.BlockSpec((BLK, D), lambda i: (i, 0)),
            pl.BlockSpec((NREL, D, D), lambda i: (0, 0, 0)),
            pl.BlockSpec((NREL, D, 16), lambda i: (0, 0, 0)),
        ],
        out_specs=pl.BlockSpec((BLK, 16 * NREL), lambda i: (i, 0)),
        out_shape=jax.ShapeDtypeStruct((N_JOB, 16 * NREL), jnp.float32),
    )(x_job, W, Ad)


def _sc_edge_ex_den(src_flat, dst_flat, asv, adv_flat, zden):
    @functools.partial(
        pl.kernel,
        out_type=[
            jax.ShapeDtypeStruct((NREL * E_PAD, H), jnp.float32),
            jax.ShapeDtypeStruct((NREL * DEN_PR, 8), jnp.float32),
            jax.ShapeDtypeStruct((NREL * H * E_PAD,), jnp.float32),
        ],
        mesh=_mesh(),
        compiler_params=_sc_params(),
        scratch_types=[
            pltpu.VMEM((N_SRC, H), jnp.float32),
            pltpu.VMEM((CH,), jnp.int32),
            pltpu.VMEM((CH,), jnp.int32),
            pltpu.VMEM((CH,), jnp.int32),
            pltpu.VMEM((CH, 16), jnp.float32),
            pltpu.VMEM((CH, H), jnp.float32),
            pltpu.VMEM((CH, 8), jnp.float32),
            pltpu.VMEM((CH, 8), jnp.float32),
            pltpu.VMEM((H, CH), jnp.float32),
            pltpu.VMEM_SHARED((DEN_PR, 8), jnp.float32),
        ],
    )
    def k(src_hbm, dst_hbm, as_hbm, ad_hbm, zden_hbm, ex_hbm, den_hbm,
          coef_hbm, as_tab, srcv, dstv, adix, adrows, exb, exb8, denrows,
          cbuf, den_sp):
        core = lax.axis_index("c")
        sub = lax.axis_index("s")
        iota16 = lax.iota(jnp.int32, 16)

        rlo = core * 2
        rhi = 2 + core * 3

        pltpu.sync_copy(zden_hbm.at[pl.ds(0, CH)], exb8)

        @pl.loop(rlo, rhi)
        def _(r):
            pltpu.sync_copy(as_hbm.at[r], as_tab)
            pltpu.sync_copy(zden_hbm, den_sp.at[pl.ds(sub * DEN_PSLAB, DEN_PSLAB)])
            plsc.subcore_barrier()

            @pl.loop(0, SLAB2 // CH)
            def _(chix):
                base = r * E_PAD + sub * SLAB2 + chix * CH
                pltpu.sync_copy(
                    (src_hbm.at[pl.ds(base, CH)], dst_hbm.at[pl.ds(base, CH)]),
                    (srcv, dstv))

                @pl.loop(0, CH, step=16)
                def _(i):
                    adix[pl.ds(i, 16)] = dstv[pl.ds(i, 16)] * 5 + r

                pltpu.sync_copy(ad_hbm.at[adix], adrows)

                epos0 = sub * SLAB2 + chix * CH

                @pl.loop(0, CH, step=16)
                def _(i):
                    s16 = srcv[pl.ds(i, 16)]
                    pos = iota16 + (epos0 + i)
                    mask = pos < E
                    off = iota16 + i
                    for h in range(H):
                        hvec = jnp.full((16,), h, jnp.int32)
                        a_s = plsc.load_gather(as_tab, [s16, hvec])
                        a_d = plsc.load_gather(adrows, [off, hvec])
                        s = a_s + a_d
                        s = jnp.maximum(s, 0.2 * s)
                        ex = jnp.where(mask, jnp.exp(s), 0.0)
                        plsc.store_scatter(exb, [off, hvec], ex)
                        plsc.store_scatter(exb8, [off, hvec], ex)

                pltpu.sync_copy(exb, ex_hbm.at[pl.ds(base, CH)])
                pltpu.sync_copy(exb8, den_sp.at[dstv], add=True)

            plsc.subcore_barrier()
            pltpu.sync_copy(
                den_sp.at[pl.ds(sub * DEN_PSLAB, DEN_PSLAB)],
                den_hbm.at[pl.ds(r * DEN_PR + sub * DEN_PSLAB, DEN_PSLAB)],
            )
            plsc.subcore_barrier()

            @pl.loop(0, SLAB2 // CH)
            def _(chix):
                base = r * E_PAD + sub * SLAB2 + chix * CH
                pltpu.sync_copy(
                    (dst_hbm.at[pl.ds(base, CH)], ex_hbm.at[pl.ds(base, CH)]),
                    (dstv, exb))

                @pl.loop(0, CH, step=16)
                def _(i):
                    adix[pl.ds(i, 16)] = dstv[pl.ds(i, 16)] + r * DEN_PR

                pltpu.sync_copy(den_hbm.at[adix], denrows)

                @pl.loop(0, CH, step=16)
                def _(i):
                    off = iota16 + i
                    for h in range(H):
                        hvec = jnp.full((16,), h, jnp.int32)
                        exv = plsc.load_gather(exb, [off, hvec])
                        dnv = plsc.load_gather(denrows, [off, hvec])
                        cbuf[h, pl.ds(i, 16)] = exv / (dnv + 1e-16)

                for h in range(H):
                    pltpu.sync_copy(
                        cbuf.at[h],
                        coef_hbm.at[pl.ds(
                            (r * H + h) * E_PAD + sub * SLAB2 + chix * CH, CH)],
                    )

    return k(src_flat, dst_flat, asv, adv_flat, zden)


def _sc_messages(src_flat, dst_flat, coef, hs_flat, zacc):
    CH2 = C // 2

    @functools.partial(
        pl.kernel,
        out_type=jax.ShapeDtypeStruct((2 * H * ACC_ROWS, CH2), jnp.float32),
        mesh=_mesh(),
        compiler_params=_sc_params(),
        scratch_types=[
            pltpu.VMEM((CH5,), jnp.int32),
            pltpu.VMEM((CH5,), jnp.int32),
            pltpu.VMEM((CH5,), jnp.int32),
            pltpu.VMEM((CH5,), jnp.float32),
            pltpu.VMEM((CH5, CH2), jnp.float32),
            pltpu.VMEM_SHARED((ACC_ROWS, CH2), jnp.float32),
        ],
    )
    def k(src_hbm, dst_hbm, coef_hbm, hs_hbm, zacc_hbm, num_hbm,
          srcv, dstv, hsix, coefv, hrows, acc):
        core = lax.axis_index("c")
        sub = lax.axis_index("s")
        w = core * NS + sub

        @pl.loop(0, 4)
        def _(pi):
            h = core * 2 + (pi >> 1)
            half = pi & 1
            p = core * 4 + pi
            pltpu.sync_copy(zacc_hbm, acc.at[pl.ds(sub * ACC_SLAB, ACC_SLAB)])
            plsc.subcore_barrier()

            @pl.loop(0, NREL)
            def _(r):
                @pl.loop(0, SLAB2 // CH5)
                def _(chix):
                    base = r * E_PAD + sub * SLAB2 + chix * CH5
                    cbase = (r * H + h) * E_PAD + sub * SLAB2 + chix * CH5
                    pltpu.sync_copy(
                        (src_hbm.at[pl.ds(base, CH5)],
                         dst_hbm.at[pl.ds(base, CH5)],
                         coef_hbm.at[pl.ds(cbase, CH5)]),
                        (srcv, dstv, coefv))

                    hs0 = r * N_SRC * 2 * H + h * 2 + half

                    @pl.loop(0, CH5, step=16)
                    def _(i):
                        hsix[pl.ds(i, 16)] = srcv[pl.ds(i, 16)] * (2 * H) + hs0

                    pltpu.sync_copy(hs_hbm.at[hsix], hrows)

                    @pl.loop(0, CH5, step=16)
                    def _(i):
                        c16 = coefv[pl.ds(i, 16)]
                        for j in range(16):
                            cv = c16[j]
                            hrows[i + j, :] = hrows[i + j, :] * cv

                    pltpu.sync_copy(hrows, acc.at[dstv], add=True)

            plsc.subcore_barrier()
            pltpu.sync_copy(
                acc.at[pl.ds(sub * ACC_SLAB, ACC_SLAB)],
                num_hbm.at[pl.ds(p * ACC_ROWS + sub * ACC_SLAB, ACC_SLAB)],
            )
            plsc.subcore_barrier()

    return k(src_flat, dst_flat, coef, hs_flat, zacc)


def _tc_epilogue(parts, x_job, bias, ln_gamma, ln_beta):
    BLK = 1000

    W16 = C // 2

    def body(*refs):
        nrefs = refs[:8]
        xr, br, gr, btr, outr = refs[8:]
        bsum = jnp.sum(br[...], axis=0, keepdims=True)
        phs = []
        for q, nr in enumerate(nrefs):
            ph = (nr[...] + xr[:, q * W16:(q + 1) * W16]
                  + bsum[:, q * W16:(q + 1) * W16])
            phs.append(jnp.maximum(ph, 0.0))
        s1 = phs[0].sum(-1, keepdims=True)
        for p in phs[1:]:
            s1 = s1 + p.sum(-1, keepdims=True)
        mu = s1 * (1.0 / D)
        s2 = ((phs[0] - mu) ** 2).sum(-1, keepdims=True)
        for p in phs[1:]:
            s2 = s2 + ((p - mu) ** 2).sum(-1, keepdims=True)
        rstd = lax.rsqrt(s2 * (1.0 / D) + 1e-5)
        for q, p in enumerate(phs):
            outr[:, q * W16:(q + 1) * W16] = (
                (p - mu) * rstd * gr[:, q * W16:(q + 1) * W16]
                + btr[:, q * W16:(q + 1) * W16]
            )

    return pl.pallas_call(
        body,
        grid=(N_JOB // BLK,),
        in_specs=[pl.BlockSpec((BLK, W16), lambda i: (i, 0))] * 8 + [
            pl.BlockSpec((BLK, D), lambda i: (i, 0)),
            pl.BlockSpec((NREL, D), lambda i: (0, 0)),
            pl.BlockSpec((1, D), lambda i: (0, 0)),
            pl.BlockSpec((1, D), lambda i: (0, 0)),
        ],
        out_specs=pl.BlockSpec((BLK, D), lambda i: (i, 0)),
        out_shape=jax.ShapeDtypeStruct((N_JOB, D), jnp.float32),
    )(*parts, x_job, bias, ln_gamma.reshape(1, D), ln_beta.reshape(1, D))


def kernel(x_job, x_station, x_machine, x_robot, ei_can_load, ei_loaded,
           ei_will_execute, ei_execute, ei_hold, W, att_src, att_dst, bias,
           ln_gamma, ln_beta):
    eis = [ei_can_load, ei_loaded, ei_will_execute, ei_execute, ei_hold]
    xs = jnp.stack([x_station, x_station, x_machine, x_machine, x_robot])

    srcs = [jnp.pad(ei[0].astype(jnp.int32), (0, E_PAD - E)) for ei in eis]
    dsts = [jnp.pad(ei[1].astype(jnp.int32), (0, E_PAD - E)) for ei in eis]
    src_flat = jnp.concatenate(srcs)
    dst_flat = jnp.concatenate(dsts)

    eye = jnp.eye(H, dtype=jnp.float32)
    As = (att_src[:, :, :, None] * eye[:, None, :]).reshape(NREL, D, H)
    Ad4 = (att_dst[:, :, :, None] * eye[:, None, :]).reshape(NREL, D, H)
    Ad = jnp.concatenate([Ad4, jnp.zeros((NREL, D, 12), jnp.float32)], axis=-1)

    hs, asv = _tc_project(xs, W, As)
    adv = _tc_dst_alpha(x_job, W, Ad)

    zden = jnp.zeros((DEN_PSLAB, 8), jnp.float32)
    ex, den, coef = _sc_edge_ex_den(
        src_flat, dst_flat, asv, adv.reshape(NREL * N_JOB, 16), zden)

    zacc = jnp.zeros((ACC_SLAB, C // 2), jnp.float32)
    num = _sc_messages(
        src_flat, dst_flat, coef, hs.reshape(NREL * N_SRC * 2 * H, C // 2),
        zacc)

    parts = [lax.slice(num, (p * ACC_ROWS, 0), (p * ACC_ROWS + N_JOB, C // 2))
             for p in range(2 * H)]
    return _tc_epilogue(parts, x_job, bias, ln_gamma, ln_beta)
